# Initial kernel scaffold; baseline (speedup 1.0000x reference)
import jax
import jax.numpy as jnp
from jax.experimental import pallas as pl


def kernel(config_logits, kernel):
    raise NotImplementedError("write your pallas kernel here")


# speedup vs baseline: 2.2454x; 2.2454x over previous
import functools

import jax
import jax.numpy as jnp
from jax import lax
from jax.experimental import pallas as pl
from jax.experimental.pallas import tpu as pltpu
from jax.experimental.pallas import tpu_sc as plsc

E = 64
B = 4
D_OUT = 1024
D_IN = 1024
D = D_OUT * D_IN

L = 16
NC = 2
NS = 16
NW = NC * NS
WPB = NW // B
PART = D // WPB
R = 1024
ROWS_PER_E = D // R
GROUP = L * R
G = PART // GROUP


def _shuf_max(v, sbuf, iota):
    for sh in (1, 2, 4, 8):
        sbuf[...] = v
        v = jnp.maximum(v, plsc.load_gather(sbuf, [iota ^ sh]))
    return v


def _shuf_min_i32(v, sbuf, iota):
    for sh in (1, 2, 4, 8):
        sbuf[...] = v
        v = jnp.minimum(v, plsc.load_gather(sbuf, [iota ^ sh]))
    return v


def _routing(lbuf, fsc, isc, iota):
    vs = [lbuf[pl.ds(j * L, L)] for j in range(E // L)]

    m = vs[0]
    for v in vs[1:]:
        m = jnp.maximum(m, v)
    m1v = _shuf_max(m, fsc, iota)

    cmin = jnp.full((L,), E, jnp.int32)
    for j, v in enumerate(vs):
        cmin = jnp.minimum(cmin, jnp.where(v == m1v, iota + (j * L), E))
    i1v = _shuf_min_i32(cmin, isc, iota)

    neg_inf = jnp.float32(-jnp.inf)
    vs2 = [jnp.where(iota + (j * L) == i1v, neg_inf, v) for j, v in enumerate(vs)]
    m2 = vs2[0]
    for v in vs2[1:]:
        m2 = jnp.maximum(m2, v)
    m2v = _shuf_max(m2, fsc, iota)

    cmin2 = jnp.full((L,), E, jnp.int32)
    for j, v in enumerate(vs2):
        cmin2 = jnp.minimum(cmin2, jnp.where(v == m2v, iota + (j * L), E))
    i2v = _shuf_min_i32(cmin2, isc, iota)

    ev = jnp.exp(m2v - m1v)
    w1v = 1.0 / (1.0 + ev)
    w2v = ev * w1v
    return i1v, i2v, w1v, w2v


def _sc_body(cl_hbm, k_hbm, out_hbm,
             lbuf, fsc, isc, idx_a, idx_b, xa, xb, obuf, sem_a, sem_b):
    wid = lax.axis_index("s") * NC + lax.axis_index("c")
    b = wid // WPB
    part = wid & (WPB - 1)

    pltpu.sync_copy(cl_hbm.at[pl.ds(b * E, E)], lbuf)
    iota = lax.iota(jnp.int32, L)
    i1v, i2v, w1v, w2v = _routing(lbuf, fsc, isc, iota)

    row_a0 = i1v * ROWS_PER_E + part * (PART // R) + iota
    row_b0 = i2v * ROWS_PER_E + part * (PART // R) + iota
    base_out = b * D + part * PART

    def group_body(g, _):
        idx_a[...] = row_a0 + g * L
        idx_b[...] = row_b0 + g * L
        ca = pltpu.async_copy(k_hbm.at[idx_a], xa, sem_a)
        cb = pltpu.async_copy(k_hbm.at[idx_b], xb, sem_b)
        ca.wait()
        cb.wait()

        def row_body(r, _):
            def vec_body(c, _):
                a0 = xa[r, pl.ds(c * L, L)]
                a1 = xb[r, pl.ds(c * L, L)]
                obuf[pl.ds(r * R + c * L, L)] = w1v * a0 + w2v * a1
                return 0

            lax.fori_loop(0, R // L, vec_body, 0)
            return 0

        lax.fori_loop(0, L, row_body, 0)
        pltpu.sync_copy(obuf, out_hbm.at[pl.ds(base_out + g * GROUP, GROUP)])
        return 0

    lax.fori_loop(0, G, group_body, 0)


_mesh = plsc.VectorSubcoreMesh(core_axis_name="c", subcore_axis_name="s")

_sc_call = functools.partial(
    pl.kernel,
    mesh=_mesh,
    compiler_params=pltpu.CompilerParams(needs_layout_passes=False),
    out_type=jax.ShapeDtypeStruct((B * D,), jnp.float32),
    scratch_types=[
        pltpu.VMEM((E,), jnp.float32),
        pltpu.VMEM((L,), jnp.float32),
        pltpu.VMEM((L,), jnp.int32),
        pltpu.VMEM((L,), jnp.int32),
        pltpu.VMEM((L,), jnp.int32),
        pltpu.VMEM((L, R), jnp.float32),
        pltpu.VMEM((L, R), jnp.float32),
        pltpu.VMEM((GROUP,), jnp.float32),
        pltpu.SemaphoreType.DMA,
        pltpu.SemaphoreType.DMA,
    ],
)(_sc_body)


def kernel(config_logits, kernel):
    cl_flat = config_logits.reshape(B * E)
    k_rows = kernel.reshape(E * ROWS_PER_E, R)
    out = _sc_call(cl_flat, k_rows)
    return out.reshape(B, D_OUT, D_IN)
